# Initial kernel scaffold; baseline (speedup 1.0000x reference)
import jax
import jax.numpy as jnp
from jax import lax
from jax.experimental import pallas as pl
from jax.experimental.pallas import tpu as pltpu
from jax.experimental.pallas import tpu_sc as plsc

NC, NS = 2, 16

N = 10000
E = 160000
D = 256
HD = D // 2

K = 125
NCHUNK = E // K
CPT_AGG = NCHUNK // NS
CPT_DEG = NCHUNK // (NC * NS)
RPT = N // NS
NPAD = 10240

_mesh = plsc.VectorSubcoreMesh(core_axis_name="c", subcore_axis_name="s")


def _deg_body(dst_hbm, zeros_hbm, ones_hbm, deg_out, idx_v, ones_v, acc):
    c = lax.axis_index("c")
    s = lax.axis_index("s")
    t = c * NS + s
    pltpu.sync_copy(zeros_hbm.at[pl.ds(s * 640, 640)], acc.at[pl.ds(s * 640, 640)])
    pltpu.sync_copy(ones_hbm, ones_v)
    pltpu.sync_copy(dst_hbm.at[pl.ds(t * CPT_DEG, CPT_DEG)], idx_v)
    plsc.subcore_barrier()

    def body(j, carry):
        pltpu.sync_copy(ones_v, acc.at[idx_v.at[j]], add=True)
        return carry

    lax.fori_loop(0, CPT_DEG, body, 0)
    plsc.subcore_barrier()
    pltpu.sync_copy(acc.at[pl.ds(s * 640, 640)], deg_out.at[c, pl.ds(s * 640, 640)])


_deg_call = pl.kernel(
    _deg_body,
    out_type=jax.ShapeDtypeStruct((NC, NPAD), jnp.float32),
    mesh=_mesh,
    scratch_types=[
        pltpu.VMEM((CPT_DEG, K), jnp.int32),
        pltpu.VMEM((K,), jnp.float32),
        pltpu.VMEM_SHARED((NPAD,), jnp.float32),
    ],
)


def _agg_body(g0, g1, src_hbm, dst_hbm, s_out, srcv, dstv, rows, acc):
    c = lax.axis_index("c")
    s = lax.axis_index("s")
    rbase = s * RPT

    @pl.when(c == 0)
    def _():
        pltpu.sync_copy(g0.at[pl.ds(rbase, RPT)], acc.at[pl.ds(rbase, RPT)])

    @pl.when(c == 1)
    def _():
        pltpu.sync_copy(g1.at[pl.ds(rbase, RPT)], acc.at[pl.ds(rbase, RPT)])

    pltpu.sync_copy(src_hbm.at[pl.ds(s * CPT_AGG, CPT_AGG)], srcv)
    pltpu.sync_copy(dst_hbm.at[pl.ds(s * CPT_AGG, CPT_AGG)], dstv)
    plsc.subcore_barrier()

    def body(j, carry):
        @pl.when(c == 0)
        def _():
            pltpu.sync_copy(g0.at[srcv.at[j]], rows)

        @pl.when(c == 1)
        def _():
            pltpu.sync_copy(g1.at[srcv.at[j]], rows)

        pltpu.sync_copy(rows, acc.at[dstv.at[j]], add=True)
        return carry

    lax.fori_loop(0, CPT_AGG, body, 0)
    plsc.subcore_barrier()
    pltpu.sync_copy(acc.at[pl.ds(rbase, RPT)], s_out.at[c, pl.ds(rbase, RPT)])


_agg_call = pl.kernel(
    _agg_body,
    out_type=jax.ShapeDtypeStruct((NC, N, HD), jnp.float32),
    mesh=_mesh,
    scratch_types=[
        pltpu.VMEM((CPT_AGG, K), jnp.int32),
        pltpu.VMEM((CPT_AGG, K), jnp.int32),
        pltpu.VMEM((K, HD), jnp.float32),
        pltpu.VMEM_SHARED((N, HD), jnp.float32),
    ],
)


R = 1000


def _mm1_body(deg_ref, x_ref, w_ref, g_ref):
    di = lax.rsqrt(deg_ref[0, :] + deg_ref[1, :] + 1.0)
    g_ref[0] = (
        jnp.dot(x_ref[...], w_ref[...], preferred_element_type=jnp.float32)
        * di[:, None]
    )


_mm1_call = pl.pallas_call(
    _mm1_body,
    grid=(N // R, NC),
    in_specs=[
        pl.BlockSpec((NC, R), lambda r, c: (0, r)),
        pl.BlockSpec((R, D), lambda r, c: (r, 0)),
        pl.BlockSpec((D, HD), lambda r, c: (0, c)),
    ],
    out_specs=pl.BlockSpec((1, R, HD), lambda r, c: (c, r, 0)),
    out_shape=jax.ShapeDtypeStruct((NC, N, HD), jnp.float32),
)


def _mm2_body(deg_ref, s_ref, b_ref, w_ref, g_ref):
    di = lax.rsqrt(deg_ref[0, :] + deg_ref[1, :] + 1.0)
    scat = jnp.concatenate([s_ref[0], s_ref[1]], axis=1)
    h1 = jnp.maximum(scat * di[:, None] + b_ref[...], 0.0)
    g_ref[0] = (
        jnp.dot(h1, w_ref[...], preferred_element_type=jnp.float32) * di[:, None]
    )


_mm2_call = pl.pallas_call(
    _mm2_body,
    grid=(N // R, NC),
    in_specs=[
        pl.BlockSpec((NC, R), lambda r, c: (0, r)),
        pl.BlockSpec((NC, R, HD), lambda r, c: (0, r, 0)),
        pl.BlockSpec((1, D), lambda r, c: (0, 0)),
        pl.BlockSpec((D, HD), lambda r, c: (0, c)),
    ],
    out_specs=pl.BlockSpec((1, R, HD), lambda r, c: (c, r, 0)),
    out_shape=jax.ShapeDtypeStruct((NC, N, HD), jnp.float32),
)


def _out_body(deg_ref, s_ref, b_ref, o_ref):
    di = lax.rsqrt(deg_ref[0, :] + deg_ref[1, :] + 1.0)
    scat = jnp.concatenate([s_ref[0], s_ref[1]], axis=1)
    o_ref[...] = scat * di[:, None] + b_ref[...]


_out_call = pl.pallas_call(
    _out_body,
    grid=(N // R,),
    in_specs=[
        pl.BlockSpec((NC, R), lambda r: (0, r)),
        pl.BlockSpec((NC, R, HD), lambda r: (0, r, 0)),
        pl.BlockSpec((1, D), lambda r: (0, 0)),
    ],
    out_specs=pl.BlockSpec((R, D), lambda r: (r, 0)),
    out_shape=jax.ShapeDtypeStruct((N, D), jnp.float32),
)


def kernel(x, edge_index, W1, b1, W2, b2):
    src = edge_index[0].reshape(NCHUNK, K)
    dst = edge_index[1].reshape(NCHUNK, K)
    zeros = jnp.zeros((NPAD,), jnp.float32)
    ones = jnp.ones((K,), jnp.float32)

    deg2 = _deg_call(dst, zeros, ones)
    g1 = _mm1_call(deg2, x, W1)
    s1 = _agg_call(g1[0], g1[1], src, dst)
    g2 = _mm2_call(deg2, s1, b1.reshape(1, D), W2)
    s2 = _agg_call(g2[0], g2[1], src, dst)
    return _out_call(deg2, s2, b2.reshape(1, D))


# speedup vs baseline: 12.9921x; 12.9921x over previous
import jax
import jax.numpy as jnp
from jax import lax
from jax.experimental import pallas as pl
from jax.experimental.pallas import tpu as pltpu
from jax.experimental.pallas import tpu_sc as plsc

NC, NS = 2, 16

N = 10000
E = 160000
D = 256
HD = D // 2

K = 125
NCHUNK = E // K
CPT_AGG = NCHUNK // NS
CPT_DEG = NCHUNK // (NC * NS)
NPAD = 10240
RPT = NPAD // NS

_mesh = plsc.VectorSubcoreMesh(
    core_axis_name="c", subcore_axis_name="s", num_cores=NC, num_subcores=NS
)


def _deg_body(dst_hbm, zeros_hbm, ones_hbm, deg_out, idx_v, ones_v, acc):
    c = lax.axis_index("c")
    s = lax.axis_index("s")
    t = c * NS + s
    pltpu.sync_copy(zeros_hbm.at[pl.ds(s * 640, 640)], acc.at[pl.ds(s * 640, 640)])
    pltpu.sync_copy(ones_hbm, ones_v)
    pltpu.sync_copy(dst_hbm.at[pl.ds(t * CPT_DEG, CPT_DEG)], idx_v)
    plsc.subcore_barrier()

    def body(j, carry):
        pltpu.sync_copy(ones_v, acc.at[idx_v.at[j]], add=True)
        return carry

    lax.fori_loop(0, CPT_DEG, body, 0)
    plsc.subcore_barrier()
    pltpu.sync_copy(acc.at[pl.ds(s * 640, 640)], deg_out.at[c, pl.ds(s * 640, 640)])


_deg_call = pl.kernel(
    _deg_body,
    out_type=jax.ShapeDtypeStruct((NC, NPAD), jnp.float32),
    mesh=_mesh,
    scratch_types=[
        pltpu.VMEM((CPT_DEG, K), jnp.int32),
        pltpu.VMEM((K,), jnp.float32),
        pltpu.VMEM_SHARED((NPAD,), jnp.float32),
    ],
)


def _agg_body(g0, g1, src_hbm, dst_hbm, s_out, srcv, dstv, rows, acc):
    c = lax.axis_index("c")
    s = lax.axis_index("s")
    rbase = s * RPT

    @pl.when(c == 0)
    def _():
        pltpu.sync_copy(g0.at[pl.ds(rbase, RPT)], acc.at[pl.ds(rbase, RPT)])

    @pl.when(c == 1)
    def _():
        pltpu.sync_copy(g1.at[pl.ds(rbase, RPT)], acc.at[pl.ds(rbase, RPT)])

    pltpu.sync_copy(src_hbm.at[pl.ds(s * CPT_AGG, CPT_AGG)], srcv)
    pltpu.sync_copy(dst_hbm.at[pl.ds(s * CPT_AGG, CPT_AGG)], dstv)
    plsc.subcore_barrier()

    def body(j, carry):
        @pl.when(c == 0)
        def _():
            pltpu.sync_copy(g0.at[srcv.at[j]], rows)

        @pl.when(c == 1)
        def _():
            pltpu.sync_copy(g1.at[srcv.at[j]], rows)

        pltpu.sync_copy(rows, acc.at[dstv.at[j]], add=True)
        return carry

    lax.fori_loop(0, CPT_AGG, body, 0)
    plsc.subcore_barrier()
    pltpu.sync_copy(acc.at[pl.ds(rbase, RPT)], s_out.at[c, pl.ds(rbase, RPT)])


_agg_call = pl.kernel(
    _agg_body,
    out_type=jax.ShapeDtypeStruct((NC, NPAD, HD), jnp.float32),
    mesh=_mesh,
    scratch_types=[
        pltpu.VMEM((CPT_AGG, K), jnp.int32),
        pltpu.VMEM((CPT_AGG, K), jnp.int32),
        pltpu.VMEM((K, HD), jnp.float32),
        pltpu.VMEM_SHARED((NPAD, HD), jnp.float32),
    ],
)


R = 1024
RO = 1000


def _dis_block(deg_ref, r):
    sl = pl.ds(r * R, R)
    return lax.rsqrt(deg_ref[0, sl] + deg_ref[1, sl] + 1.0)


def _mm1_body(deg_ref, x_ref, w_ref, g_ref, dis_ref):
    di = _dis_block(deg_ref, pl.program_id(0))
    dis_ref[...] = di[:, None]
    g_ref[0] = (
        jnp.dot(x_ref[...], w_ref[...], preferred_element_type=jnp.float32)
        * di[:, None]
    )


_mm1_call = pl.pallas_call(
    _mm1_body,
    grid=(NPAD // R, NC),
    in_specs=[
        pl.BlockSpec((NC, NPAD), lambda r, c: (0, 0)),
        pl.BlockSpec((R, D), lambda r, c: (r, 0)),
        pl.BlockSpec((D, HD), lambda r, c: (0, c)),
    ],
    out_specs=[
        pl.BlockSpec((1, R, HD), lambda r, c: (c, r, 0)),
        pl.BlockSpec((R, 1), lambda r, c: (r, 0)),
    ],
    out_shape=[
        jax.ShapeDtypeStruct((NC, NPAD, HD), jnp.float32),
        jax.ShapeDtypeStruct((NPAD, 1), jnp.float32),
    ],
)


def _mm2_body(dis_ref, s_ref, b_ref, w_ref, g_ref):
    di = dis_ref[...]
    scat = jnp.concatenate([s_ref[0], s_ref[1]], axis=1)
    h1 = jnp.maximum(scat * di + b_ref[...], 0.0)
    g_ref[0] = jnp.dot(h1, w_ref[...], preferred_element_type=jnp.float32) * di


_mm2_call = pl.pallas_call(
    _mm2_body,
    grid=(NPAD // R, NC),
    in_specs=[
        pl.BlockSpec((R, 1), lambda r, c: (r, 0)),
        pl.BlockSpec((NC, R, HD), lambda r, c: (0, r, 0)),
        pl.BlockSpec((1, D), lambda r, c: (0, 0)),
        pl.BlockSpec((D, HD), lambda r, c: (0, c)),
    ],
    out_specs=pl.BlockSpec((1, R, HD), lambda r, c: (c, r, 0)),
    out_shape=jax.ShapeDtypeStruct((NC, NPAD, HD), jnp.float32),
)


def _out_body(dis_ref, s_ref, b_ref, o_ref):
    scat = jnp.concatenate([s_ref[0], s_ref[1]], axis=1)
    o_ref[...] = scat * dis_ref[...] + b_ref[...]


_out_call = pl.pallas_call(
    _out_body,
    grid=(N // RO,),
    in_specs=[
        pl.BlockSpec((RO, 1), lambda r: (r, 0)),
        pl.BlockSpec((NC, RO, HD), lambda r: (0, r, 0)),
        pl.BlockSpec((1, D), lambda r: (0, 0)),
    ],
    out_specs=pl.BlockSpec((RO, D), lambda r: (r, 0)),
    out_shape=jax.ShapeDtypeStruct((N, D), jnp.float32),
)


def kernel(x, edge_index, W1, b1, W2, b2):
    src = edge_index[0].reshape(NCHUNK, K)
    dst = edge_index[1].reshape(NCHUNK, K)
    zeros = jnp.zeros((NPAD,), jnp.float32)
    ones = jnp.ones((K,), jnp.float32)

    deg2 = _deg_call(dst, zeros, ones)
    g1, dis = _mm1_call(deg2, x, W1)
    s1 = _agg_call(g1[0], g1[1], src, dst)
    g2 = _mm2_call(dis, s1, b1.reshape(1, D), W2)
    s2 = _agg_call(g2[0], g2[1], src, dst)
    return _out_call(dis, s2, b2.reshape(1, D))


# speedup vs baseline: 13.4606x; 1.0361x over previous
import jax
import jax.numpy as jnp
from jax import lax
from jax.experimental import pallas as pl
from jax.experimental.pallas import tpu as pltpu
from jax.experimental.pallas import tpu_sc as plsc

NC, NS = 2, 16

N = 10000
E = 160000
D = 256
HD = D // 2

K = 125
NCHUNK = E // K
CPT_AGG = NCHUNK // NS
CPT_DEG = NCHUNK // (NC * NS)
NPAD = 10240
RPT = NPAD // NS

_mesh = plsc.VectorSubcoreMesh(
    core_axis_name="c", subcore_axis_name="s", num_cores=NC, num_subcores=NS
)


def _deg_body(dst_hbm, zeros_hbm, ones_hbm, deg_out, idx_v, ones_v, acc):
    c = lax.axis_index("c")
    s = lax.axis_index("s")
    t = c * NS + s
    pltpu.sync_copy(zeros_hbm.at[pl.ds(s * 640, 640)], acc.at[pl.ds(s * 640, 640)])
    pltpu.sync_copy(ones_hbm, ones_v)
    pltpu.sync_copy(dst_hbm.at[pl.ds(t * CPT_DEG, CPT_DEG)], idx_v)
    plsc.subcore_barrier()

    def body(j, carry):
        pltpu.sync_copy(ones_v, acc.at[idx_v.at[j]], add=True)
        return carry

    lax.fori_loop(0, CPT_DEG, body, 0)
    plsc.subcore_barrier()
    pltpu.sync_copy(acc.at[pl.ds(s * 640, 640)], deg_out.at[c, pl.ds(s * 640, 640)])


_deg_call = pl.kernel(
    _deg_body,
    out_type=jax.ShapeDtypeStruct((NC, NPAD), jnp.float32),
    mesh=_mesh,
    scratch_types=[
        pltpu.VMEM((CPT_DEG, K), jnp.int32),
        pltpu.VMEM((K,), jnp.float32),
        pltpu.VMEM_SHARED((NPAD,), jnp.float32),
    ],
)


def _agg_body(g3, src_hbm, dst_hbm, s_out, srcv, dstv, rows0, acc):
    c = lax.axis_index("c")
    s = lax.axis_index("s")
    g = g3.at[c]
    rbase = s * RPT

    pltpu.sync_copy(g3.at[c, pl.ds(rbase, RPT)], acc.at[pl.ds(rbase, RPT)])
    pltpu.sync_copy(src_hbm.at[pl.ds(s * CPT_AGG, CPT_AGG)], srcv)
    pltpu.sync_copy(dst_hbm.at[pl.ds(s * CPT_AGG, CPT_AGG)], dstv)
    plsc.subcore_barrier()

    def body(j, carry):
        pltpu.sync_copy(g.at[srcv.at[j]], rows0)
        pltpu.sync_copy(rows0, acc.at[dstv.at[j]], add=True)
        return carry

    lax.fori_loop(0, CPT_AGG, body, 0)
    plsc.subcore_barrier()
    pltpu.sync_copy(acc.at[pl.ds(rbase, RPT)], s_out.at[c, pl.ds(rbase, RPT)])


_agg_call = pl.kernel(
    _agg_body,
    out_type=jax.ShapeDtypeStruct((NC, NPAD, HD), jnp.float32),
    mesh=_mesh,
    scratch_types=[
        pltpu.VMEM((CPT_AGG, K), jnp.int32),
        pltpu.VMEM((CPT_AGG, K), jnp.int32),
        pltpu.VMEM((K, HD), jnp.float32),
        pltpu.VMEM_SHARED((NPAD, HD), jnp.float32),
    ],
)


R = 1024
RO = 1000


def _dis_block(deg_ref, r):
    sl = pl.ds(r * R, R)
    return lax.rsqrt(deg_ref[0, sl] + deg_ref[1, sl] + 1.0)


def _mm1_body(deg_ref, x_ref, w_ref, g_ref, dis_ref):
    di = _dis_block(deg_ref, pl.program_id(0))
    dis_ref[...] = di[:, None]
    g_ref[0] = (
        jnp.dot(x_ref[...], w_ref[...], preferred_element_type=jnp.float32)
        * di[:, None]
    )


_mm1_call = pl.pallas_call(
    _mm1_body,
    grid=(NPAD // R, NC),
    in_specs=[
        pl.BlockSpec((NC, NPAD), lambda r, c: (0, 0)),
        pl.BlockSpec((R, D), lambda r, c: (r, 0)),
        pl.BlockSpec((D, HD), lambda r, c: (0, c)),
    ],
    out_specs=[
        pl.BlockSpec((1, R, HD), lambda r, c: (c, r, 0)),
        pl.BlockSpec((R, 1), lambda r, c: (r, 0)),
    ],
    out_shape=[
        jax.ShapeDtypeStruct((NC, NPAD, HD), jnp.float32),
        jax.ShapeDtypeStruct((NPAD, 1), jnp.float32),
    ],
)


def _mm2_body(dis_ref, s_ref, b_ref, w_ref, g_ref):
    di = dis_ref[...]
    scat = jnp.concatenate([s_ref[0], s_ref[1]], axis=1)
    h1 = jnp.maximum(scat * di + b_ref[...], 0.0)
    g_ref[0] = jnp.dot(h1, w_ref[...], preferred_element_type=jnp.float32) * di


_mm2_call = pl.pallas_call(
    _mm2_body,
    grid=(NPAD // R, NC),
    in_specs=[
        pl.BlockSpec((R, 1), lambda r, c: (r, 0)),
        pl.BlockSpec((NC, R, HD), lambda r, c: (0, r, 0)),
        pl.BlockSpec((1, D), lambda r, c: (0, 0)),
        pl.BlockSpec((D, HD), lambda r, c: (0, c)),
    ],
    out_specs=pl.BlockSpec((1, R, HD), lambda r, c: (c, r, 0)),
    out_shape=jax.ShapeDtypeStruct((NC, NPAD, HD), jnp.float32),
)


def _out_body(dis_ref, s_ref, b_ref, o_ref):
    scat = jnp.concatenate([s_ref[0], s_ref[1]], axis=1)
    o_ref[...] = scat * dis_ref[...] + b_ref[...]


_out_call = pl.pallas_call(
    _out_body,
    grid=(N // RO,),
    in_specs=[
        pl.BlockSpec((RO, 1), lambda r: (r, 0)),
        pl.BlockSpec((NC, RO, HD), lambda r: (0, r, 0)),
        pl.BlockSpec((1, D), lambda r: (0, 0)),
    ],
    out_specs=pl.BlockSpec((RO, D), lambda r: (r, 0)),
    out_shape=jax.ShapeDtypeStruct((N, D), jnp.float32),
)


def kernel(x, edge_index, W1, b1, W2, b2):
    src = edge_index[0].reshape(NCHUNK, K)
    dst = edge_index[1].reshape(NCHUNK, K)
    zeros = jnp.zeros((NPAD,), jnp.float32)
    ones = jnp.ones((K,), jnp.float32)

    deg2 = _deg_call(dst, zeros, ones)
    g1, dis = _mm1_call(deg2, x, W1)
    s1 = _agg_call(g1, src, dst)
    g2 = _mm2_call(dis, s1, b1.reshape(1, D), W2)
    s2 = _agg_call(g2, src, dst)
    return _out_call(dis, s2, b2.reshape(1, D))


# speedup vs baseline: 14.0754x; 1.0457x over previous
import jax
import jax.numpy as jnp
from jax import lax
from jax.experimental import pallas as pl
from jax.experimental.pallas import tpu as pltpu
from jax.experimental.pallas import tpu_sc as plsc

NC, NS = 2, 16

N = 10000
E = 160000
D = 256
HD = D // 2

K = 125
NCHUNK = E // K
CPT_AGG = NCHUNK // NS
CPT_DEG = NCHUNK // (NC * NS)
NPAD = 10240
RPT = NPAD // NS

_mesh = plsc.VectorSubcoreMesh(
    core_axis_name="c", subcore_axis_name="s", num_cores=NC, num_subcores=NS
)


def _deg_body(dst_hbm, zeros_hbm, ones_hbm, deg_out, idx_v, ones_v, acc):
    c = lax.axis_index("c")
    s = lax.axis_index("s")
    t = c * NS + s
    pltpu.sync_copy(zeros_hbm.at[pl.ds(s * 640, 640)], acc.at[pl.ds(s * 640, 640)])
    pltpu.sync_copy(ones_hbm, ones_v)
    pltpu.sync_copy(dst_hbm.at[pl.ds(t * CPT_DEG, CPT_DEG)], idx_v)
    plsc.subcore_barrier()

    def body(j, carry):
        pltpu.sync_copy(ones_v, acc.at[idx_v.at[j]], add=True)
        return carry

    lax.fori_loop(0, CPT_DEG, body, 0)
    plsc.subcore_barrier()
    pltpu.sync_copy(acc.at[pl.ds(s * 640, 640)], deg_out.at[c, pl.ds(s * 640, 640)])


_deg_call = pl.kernel(
    _deg_body,
    out_type=jax.ShapeDtypeStruct((NC, NPAD), jnp.float32),
    mesh=_mesh,
    scratch_types=[
        pltpu.VMEM((CPT_DEG, K), jnp.int32),
        pltpu.VMEM((K,), jnp.float32),
        pltpu.VMEM_SHARED((NPAD,), jnp.float32),
    ],
)


def _agg_body(g3, src_hbm, dst_hbm, s_out, srcv, dstv, rows0, acc):
    c = lax.axis_index("c")
    s = lax.axis_index("s")
    g = g3.at[c]
    rbase = s * RPT

    pltpu.sync_copy(g3.at[c, pl.ds(rbase, RPT)], acc.at[pl.ds(rbase, RPT)])
    pltpu.sync_copy(src_hbm.at[pl.ds(s * CPT_AGG, CPT_AGG)], srcv)
    pltpu.sync_copy(dst_hbm.at[pl.ds(s * CPT_AGG, CPT_AGG)], dstv)
    plsc.subcore_barrier()

    def body(j, carry):
        pltpu.sync_copy(g.at[srcv.at[j]], rows0)
        pltpu.sync_copy(rows0, acc.at[dstv.at[j]], add=True)
        return carry

    lax.fori_loop(0, CPT_AGG, body, 0)
    plsc.subcore_barrier()
    pltpu.sync_copy(acc.at[pl.ds(rbase, RPT)], s_out.at[c, pl.ds(rbase, RPT)])


_agg_call = pl.kernel(
    _agg_body,
    out_type=jax.ShapeDtypeStruct((NC, NPAD, HD), jnp.float32),
    mesh=_mesh,
    scratch_types=[
        pltpu.VMEM((CPT_AGG, K), jnp.int32),
        pltpu.VMEM((CPT_AGG, K), jnp.int32),
        pltpu.VMEM((K, HD), jnp.float32),
        pltpu.VMEM_SHARED((NPAD, HD), jnp.float32),
    ],
)


R = 1024
RO = 1000


def _dis_block(deg_ref, r):
    sl = pl.ds(r * R, R)
    return lax.rsqrt(deg_ref[0, sl] + deg_ref[1, sl] + 1.0)


def _mm1_body(deg_ref, x_ref, w_ref, g_ref, dis_ref):
    di = _dis_block(deg_ref, pl.program_id(0))
    dis_ref[...] = di[:, None]
    h = jnp.dot(x_ref[...], w_ref[...], preferred_element_type=jnp.float32)
    h = h * di[:, None]
    g_ref[0] = h[:, :HD]
    g_ref[1] = h[:, HD:]


_mm1_call = pl.pallas_call(
    _mm1_body,
    grid=(NPAD // R,),
    in_specs=[
        pl.BlockSpec((NC, NPAD), lambda r: (0, 0)),
        pl.BlockSpec((R, D), lambda r: (r, 0)),
        pl.BlockSpec((D, D), lambda r: (0, 0)),
    ],
    out_specs=[
        pl.BlockSpec((NC, R, HD), lambda r: (0, r, 0)),
        pl.BlockSpec((R, 1), lambda r: (r, 0)),
    ],
    out_shape=[
        jax.ShapeDtypeStruct((NC, NPAD, HD), jnp.float32),
        jax.ShapeDtypeStruct((NPAD, 1), jnp.float32),
    ],
)


def _mm2_body(dis_ref, s_ref, b_ref, w_ref, g_ref):
    di = dis_ref[...]
    scat = jnp.concatenate([s_ref[0], s_ref[1]], axis=1)
    h1 = jnp.maximum(scat * di + b_ref[...], 0.0)
    g = jnp.dot(h1, w_ref[...], preferred_element_type=jnp.float32) * di
    g_ref[0] = g[:, :HD]
    g_ref[1] = g[:, HD:]


_mm2_call = pl.pallas_call(
    _mm2_body,
    grid=(NPAD // R,),
    in_specs=[
        pl.BlockSpec((R, 1), lambda r: (r, 0)),
        pl.BlockSpec((NC, R, HD), lambda r: (0, r, 0)),
        pl.BlockSpec((1, D), lambda r: (0, 0)),
        pl.BlockSpec((D, D), lambda r: (0, 0)),
    ],
    out_specs=pl.BlockSpec((NC, R, HD), lambda r: (0, r, 0)),
    out_shape=jax.ShapeDtypeStruct((NC, NPAD, HD), jnp.float32),
)


def _out_body(dis_ref, s_ref, b_ref, o_ref):
    scat = jnp.concatenate([s_ref[0], s_ref[1]], axis=1)
    o_ref[...] = scat * dis_ref[...] + b_ref[...]


_out_call = pl.pallas_call(
    _out_body,
    grid=(N // RO,),
    in_specs=[
        pl.BlockSpec((RO, 1), lambda r: (r, 0)),
        pl.BlockSpec((NC, RO, HD), lambda r: (0, r, 0)),
        pl.BlockSpec((1, D), lambda r: (0, 0)),
    ],
    out_specs=pl.BlockSpec((RO, D), lambda r: (r, 0)),
    out_shape=jax.ShapeDtypeStruct((N, D), jnp.float32),
)


def kernel(x, edge_index, W1, b1, W2, b2):
    src = edge_index[0].reshape(NCHUNK, K)
    dst = edge_index[1].reshape(NCHUNK, K)
    zeros = jnp.zeros((NPAD,), jnp.float32)
    ones = jnp.ones((K,), jnp.float32)

    deg2 = _deg_call(dst, zeros, ones)
    g1, dis = _mm1_call(deg2, x, W1)
    s1 = _agg_call(g1, src, dst)
    g2 = _mm2_call(dis, s1, b1.reshape(1, D), W2)
    s2 = _agg_call(g2, src, dst)
    return _out_call(dis, s2, b2.reshape(1, D))
